# baseline (speedup 1.0000x reference)
import functools

import jax
import jax.numpy as jnp
from jax import lax
from jax.experimental import pallas as pl
from jax.experimental.pallas import tpu as pltpu
from jax.experimental.pallas import tpu_sc as plsc

NC = 2
NS = 16
LANES = 16
NW = NC * NS

B = 16384
K = 32
BPW = B // NW
STAGE = 128
NSTAGE = BPW // STAGE
NGRP = STAGE // LANES

_mesh = plsc.VectorSubcoreMesh(core_axis_name="c", subcore_axis_name="s")


@functools.partial(
    pl.kernel,
    out_type=jax.ShapeDtypeStruct((B,), jnp.float32),
    mesh=_mesh,
    scratch_types=[
        pltpu.VMEM((BPW,), jnp.int32),
        pltpu.VMEM((BPW,), jnp.int32),
        pltpu.VMEM((BPW,), jnp.int32),
        pltpu.VMEM((BPW,), jnp.int32),
        pltpu.VMEM((BPW,), jnp.int32),
        pltpu.VMEM((BPW,), jnp.int32),
        pltpu.VMEM((STAGE, 128), jnp.float32),
        pltpu.VMEM((STAGE, 128), jnp.float32),
        pltpu.VMEM((STAGE, 128), jnp.float32),
        pltpu.VMEM((STAGE, 128), jnp.float32),
        pltpu.VMEM((BPW,), jnp.float32),
        pltpu.SemaphoreType.DMA,
        pltpu.SemaphoreType.DMA,
    ],
    compiler_params=pltpu.CompilerParams(
        needs_layout_passes=False, use_tc_tiling_on_sc=True),
)
def _policy_lr_sc(rows_hbm, cols_hbm, lq_hbm, rq_hbm, out_hbm,
                  rows_v, cols_v, rq_v, cq_v, remr_v, remc_v,
                  lq0, lq1, rq0, rq1, res_v, sem0, sem1):
    wid = lax.axis_index("s") * NC + lax.axis_index("c")
    base = wid * BPW

    pltpu.sync_copy(rows_hbm.at[pl.ds(base, BPW)], rows_v)
    pltpu.sync_copy(cols_hbm.at[pl.ds(base, BPW)], cols_v)

    def precompute(j, carry):
        r16 = rows_v[pl.ds(j * LANES, LANES)]
        c16 = cols_v[pl.ds(j * LANES, LANES)]
        rq_v[pl.ds(j * LANES, LANES)] = r16 >> 2
        cq_v[pl.ds(j * LANES, LANES)] = c16 >> 2
        remr_v[pl.ds(j * LANES, LANES)] = (r16 & 3) << 5
        remc_v[pl.ds(j * LANES, LANES)] = (c16 & 3) << 5
        return carry

    lax.fori_loop(0, BPW // LANES, precompute, 0)

    def fire(s, lbuf, rbuf, sem):
        return [
            pltpu.async_copy(
                lq_hbm.at[rq_v.at[pl.ds(s * STAGE, STAGE)]], lbuf, sem),
            pltpu.async_copy(
                rq_hbm.at[cq_v.at[pl.ds(s * STAGE, STAGE)]], rbuf, sem),
        ]

    iota = lax.iota(jnp.int32, LANES)

    def dot(s, lbuf, rbuf):
        def dot_group(g, carry):
            b16 = g * LANES + iota
            off = s * STAGE + g * LANES
            lcol = remr_v[pl.ds(off, LANES)]
            rcol = remc_v[pl.ds(off, LANES)]
            acc = jnp.zeros((LANES,), jnp.float32)
            for k in range(K):
                lv = plsc.load_gather(lbuf, [b16, lcol + k])
                rv = plsc.load_gather(rbuf, [b16, rcol + k])
                acc = acc + lv * rv
            res_v[pl.ds(off, LANES)] = acc
            return carry

        lax.fori_loop(0, NGRP, dot_group, 0)

    bufs = [(lq0, rq0, sem0), (lq1, rq1, sem1)]
    pending = {0: fire(0, *bufs[0])}
    for s in range(NSTAGE):
        lbuf, rbuf, _ = bufs[s % 2]
        if s + 1 < NSTAGE:
            pending[s + 1] = fire(s + 1, *bufs[(s + 1) % 2])
        for cp in pending.pop(s):
            cp.wait()
        dot(s, lbuf, rbuf)

    pltpu.sync_copy(res_v, out_hbm.at[pl.ds(base, BPW)])


def kernel(rows, cols, L, R, log_sigma):
    n, k = L.shape
    m = R.shape[1]
    res = _policy_lr_sc(
        rows.astype(jnp.int32),
        cols.astype(jnp.int32),
        L.reshape(n // 4, 128),
        R.T.reshape(m // 4, 128),
    )
    return res, jnp.clip(log_sigma, -2.5, 0.0)


# speedup vs baseline: 1.6566x; 1.6566x over previous
import functools

import jax
import jax.numpy as jnp
from jax import lax
from jax.experimental import pallas as pl
from jax.experimental.pallas import tpu as pltpu
from jax.experimental.pallas import tpu_sc as plsc

NC = 2
NS = 16
LANES = 16
NW = NC * NS

B = 16384
K = 32
BPW = B // NW
PASS = 256
NPASS = BPW // PASS
NGRP = PASS // LANES

_mesh = plsc.VectorSubcoreMesh(core_axis_name="c", subcore_axis_name="s")


@functools.partial(
    pl.kernel,
    out_type=jax.ShapeDtypeStruct((B,), jnp.float32),
    mesh=_mesh,
    scratch_types=[
        pltpu.VMEM((BPW,), jnp.int32),
        pltpu.VMEM((BPW,), jnp.int32),
        pltpu.VMEM((PASS, K), jnp.float32),
        pltpu.VMEM((PASS, K), jnp.float32),
        pltpu.VMEM((BPW,), jnp.float32),
        pltpu.SemaphoreType.DMA,
        pltpu.SemaphoreType.DMA,
    ],
    compiler_params=pltpu.CompilerParams(
        needs_layout_passes=False, use_tc_tiling_on_sc=True),
)
def _policy_lr_sc(rows_hbm, cols_hbm, l_hbm, rt_hbm, out_hbm,
                  rows_v, cols_v, lg_v, rg_v, res_v, sem_l, sem_r):
    wid = lax.axis_index("s") * NC + lax.axis_index("c")
    base = wid * BPW

    pltpu.sync_copy(rows_hbm.at[pl.ds(base, BPW)], rows_v)
    pltpu.sync_copy(cols_hbm.at[pl.ds(base, BPW)], cols_v)

    iota = lax.iota(jnp.int32, LANES)

    for p in range(NPASS):
        def fire(j, carry):
            r16 = rows_v[pl.ds(p * PASS + j * LANES, LANES)]
            c16 = cols_v[pl.ds(p * PASS + j * LANES, LANES)]
            for lane in range(LANES):
                brow = j * LANES + lane
                pltpu.async_copy(
                    l_hbm.at[pl.ds(r16[lane], 1), :],
                    lg_v.at[pl.ds(brow, 1), :],
                    sem_l,
                )
                pltpu.async_copy(
                    rt_hbm.at[pl.ds(c16[lane], 1), :],
                    rg_v.at[pl.ds(brow, 1), :],
                    sem_r,
                )
            return carry

        lax.fori_loop(0, NGRP, fire, 0)

        def drain(i, carry):
            pltpu.make_async_copy(
                l_hbm.at[pl.ds(0, 1), :], lg_v.at[pl.ds(0, 1), :],
                sem_l).wait()
            pltpu.make_async_copy(
                rt_hbm.at[pl.ds(0, 1), :], rg_v.at[pl.ds(0, 1), :],
                sem_r).wait()
            return carry

        lax.fori_loop(0, PASS, drain, 0)

        def dot_group(g, carry):
            b16 = g * LANES + iota
            acc = jnp.zeros((LANES,), jnp.float32)
            for k in range(K):
                kvec = jnp.full((LANES,), k, jnp.int32)
                lv = plsc.load_gather(lg_v, [b16, kvec])
                rv = plsc.load_gather(rg_v, [b16, kvec])
                acc = acc + lv * rv
            res_v[pl.ds(p * PASS + g * LANES, LANES)] = acc
            return carry

        lax.fori_loop(0, NGRP, dot_group, 0)

    pltpu.sync_copy(res_v, out_hbm.at[pl.ds(base, BPW)])


def kernel(rows, cols, L, R, log_sigma):
    res = _policy_lr_sc(
        rows.astype(jnp.int32),
        cols.astype(jnp.int32),
        L,
        R.T,
    )
    return res, jnp.clip(log_sigma, -2.5, 0.0)
